# baseline (speedup 1.0000x reference)
import functools

import jax
import jax.numpy as jnp
from jax import lax
from jax.experimental import pallas as pl
from jax.experimental.pallas import tpu as pltpu
from jax.experimental.pallas import tpu_sc as plsc

_NUM_EMB = 1000000
_D = 32
_B = 16384
_H = 200

_TOT = _B * _H
_NC, _NS = 2, 16
_NW = _NC * _NS
_BT = _B // 128
_NTILE = _H * _BT
_PER_W = _NTILE // _NW
_NBUF = 4
_ROUNDS = _PER_W // _NBUF
_CHUNK = 80
_CIDX = _CHUNK * 128


@functools.partial(
    pl.kernel,
    mesh=plsc.VectorSubcoreMesh(core_axis_name="c", subcore_axis_name="s"),
    out_type=jax.ShapeDtypeStruct((_H, _D // 8, _BT, 8, 128), jnp.float32),
    compiler_params=pltpu.CompilerParams(
        use_tc_tiling_on_sc=False, needs_layout_passes=False
    ),
    scratch_types=[
        pltpu.VMEM((2 * _CIDX + 16,), jnp.int32),
        pltpu.VMEM((_NBUF, 128, _D), jnp.float32),
        pltpu.VMEM((_NBUF, _D, 137), jnp.float32),
    ]
    + [pltpu.SemaphoreType.DMA] * (2 * _NBUF),
)
def _emb_gather(idx_hbm, tab_hbm, out_hbm, idx_v, rows_v, trans_v, *sems):
    sem_g = sems[:_NBUF]
    sem_o = sems[_NBUF:]
    wid = lax.axis_index("s") * _NC + lax.axis_index("c")
    base = wid * _PER_W

    idx_v[pl.ds(2 * _CIDX, 16)] = jnp.arange(16, dtype=jnp.int32)

    def load_chunk(g):
        c = g // _CHUNK
        src = pl.multiple_of((base + c * _CHUNK) * 128, 1024)
        dst = pl.multiple_of((c % 2) * _CIDX, 1024)
        pltpu.sync_copy(
            idx_hbm.at[pl.ds(src, _CIDX)], idx_v.at[pl.ds(dst, _CIDX)]
        )

    def fire_gather(g, b):
        off = pl.multiple_of((g % (2 * _CHUNK)) * 128, 128)
        pltpu.async_copy(
            tab_hbm.at[idx_v.at[pl.ds(off, 128)]], rows_v.at[b], sem_g[b]
        )

    def drain_gather(b):
        pltpu.make_async_copy(
            tab_hbm.at[idx_v.at[pl.ds(0, 128)]], rows_v.at[b], sem_g[b]
        ).wait()

    def transpose_block(b, r_lo, r_hi, czero):
        tr_ref = trans_v.at[b]
        cj = czero
        one = czero + 1
        for j in range(128):
            lo = rows_v[b, j, pl.ds(0, 16)]
            hi = rows_v[b, j, pl.ds(16, 16)]
            plsc.store_scatter(tr_ref, [r_lo, cj], lo)
            plsc.store_scatter(tr_ref, [r_hi, cj], hi)
            cj = cj + one

    def fire_stores(gp, b):
        g_abs = base + gp
        h = g_abs // _BT
        tc = g_abs % _BT
        for tr in range(_D // 8):
            pltpu.async_copy(
                trans_v.at[b, pl.ds(tr * 8, 8), pl.ds(0, 128)],
                out_hbm.at[h, tr, tc],
                sem_o[b],
            )

    def drain_stores(b):
        for _ in range(_D // 8):
            pltpu.make_async_copy(
                trans_v.at[b, pl.ds(0, 8), pl.ds(0, 128)],
                out_hbm.at[0, 0, 0],
                sem_o[b],
            ).wait()

    def body(r, carry):
        g0 = r * _NBUF
        r_lo = idx_v[pl.ds(2 * _CIDX, 16)]
        czero = jax.lax.shift_right_logical(r_lo, 31)
        r_hi = r_lo + 16

        @pl.when(g0 % _CHUNK == 0)
        def _():
            load_chunk(g0)

        for vb in range(_NBUF):
            g = g0 + vb
            gp = g - _NBUF

            @pl.when(g >= 2 * _NBUF)
            def _():
                drain_stores(vb)

            @pl.when(g >= _NBUF)
            def _():
                drain_gather(vb)
                transpose_block(vb, r_lo, r_hi, czero)
                fire_stores(gp, vb)

            fire_gather(g, vb)
        return carry

    lax.fori_loop(0, _ROUNDS, body, 0)

    for vb in range(_NBUF):
        gp = _PER_W - _NBUF + vb
        drain_stores(vb)
        drain_gather(vb)
        ep_lo = idx_v[pl.ds(2 * _CIDX, 16)]
        transpose_block(
            vb, ep_lo, ep_lo + 16, jax.lax.shift_right_logical(ep_lo, 31)
        )
        fire_stores(gp, vb)
    for vb in range(_NBUF):
        drain_stores(vb)


def kernel(token_ids, weight):
    idx_hm = jnp.transpose(token_ids).reshape(_TOT)
    out5 = _emb_gather(idx_hm, weight)
    return out5.transpose(2, 4, 0, 1, 3).reshape(_B, _H, _D)


# speedup vs baseline: 1.0096x; 1.0096x over previous
import functools

import jax
import jax.numpy as jnp
from jax import lax
from jax.experimental import pallas as pl
from jax.experimental.pallas import tpu as pltpu
from jax.experimental.pallas import tpu_sc as plsc

_NUM_EMB = 1000000
_D = 32
_B = 16384
_H = 200

_TOT = _B * _H
_NC, _NS = 2, 16
_NW = _NC * _NS
_BT = _B // 128
_NTILE = _H * _BT
_PER_W = _NTILE // _NW
_NBUF = 4
_ROUNDS = _PER_W // _NBUF
_CHUNK = 80
_CIDX = _CHUNK * 128


@functools.partial(
    pl.kernel,
    mesh=plsc.VectorSubcoreMesh(core_axis_name="c", subcore_axis_name="s"),
    out_type=jax.ShapeDtypeStruct((_H, _D // 8, _BT, 8, 128), jnp.float32),
    compiler_params=pltpu.CompilerParams(
        use_tc_tiling_on_sc=False, needs_layout_passes=False
    ),
    scratch_types=[
        pltpu.VMEM((2 * _CIDX + 16,), jnp.int32),
        pltpu.VMEM((_NBUF, 128, _D), jnp.float32),
        pltpu.VMEM((_NBUF, _D, 137), jnp.float32),
    ]
    + [pltpu.SemaphoreType.DMA] * (2 * _NBUF),
)
def _emb_gather(idx_hbm, tab_hbm, out_hbm, idx_v, rows_v, trans_v, *sems):
    sem_g = sems[:_NBUF]
    sem_o = sems[_NBUF:]
    wid = lax.axis_index("s") * _NC + lax.axis_index("c")
    base = wid * _PER_W

    idx_v[pl.ds(2 * _CIDX, 16)] = jnp.arange(16, dtype=jnp.int32)

    def load_chunk(g):
        c = g // _CHUNK
        src = pl.multiple_of((base + c * _CHUNK) * 128, 1024)
        dst = pl.multiple_of((c % 2) * _CIDX, 1024)
        pltpu.sync_copy(
            idx_hbm.at[pl.ds(src, _CIDX)], idx_v.at[pl.ds(dst, _CIDX)]
        )

    def fire_gather(g, b):
        off = pl.multiple_of((g % (2 * _CHUNK)) * 128, 128)
        pltpu.async_copy(
            tab_hbm.at[idx_v.at[pl.ds(off, 128)]], rows_v.at[b], sem_g[b]
        )

    def drain_gather(b):
        pltpu.make_async_copy(
            tab_hbm.at[idx_v.at[pl.ds(0, 128)]], rows_v.at[b], sem_g[b]
        ).wait()

    def transpose_block(b, r_lo, r_hi, czero):
        tr_ref = trans_v.at[b]
        for j in range(128):
            lo = rows_v[b, j, pl.ds(0, 16)]
            hi = rows_v[b, j, pl.ds(16, 16)]
            cj = czero + j
            plsc.store_scatter(tr_ref, [r_lo, cj], lo)
            plsc.store_scatter(tr_ref, [r_hi, cj], hi)

    def fire_stores(gp, b):
        g_abs = base + gp
        h = g_abs // _BT
        tc = g_abs % _BT
        for tr in range(_D // 8):
            pltpu.async_copy(
                trans_v.at[b, pl.ds(tr * 8, 8), pl.ds(0, 128)],
                out_hbm.at[h, tr, tc],
                sem_o[b],
            )

    def drain_stores(b):
        for _ in range(_D // 8):
            pltpu.make_async_copy(
                trans_v.at[b, pl.ds(0, 8), pl.ds(0, 128)],
                out_hbm.at[0, 0, 0],
                sem_o[b],
            ).wait()

    def body(r, carry):
        g0 = r * _NBUF
        r_lo = idx_v[pl.ds(2 * _CIDX, 16)]
        czero = jax.lax.shift_right_logical(r_lo, 31)
        r_hi = r_lo + 16

        @pl.when(g0 % _CHUNK == 0)
        def _():
            load_chunk(g0)

        for vb in range(_NBUF):
            g = g0 + vb
            gp = g - _NBUF

            @pl.when(g >= 2 * _NBUF)
            def _():
                drain_stores(vb)

            @pl.when(g >= _NBUF)
            def _():
                drain_gather(vb)
                transpose_block(vb, r_lo, r_hi, czero)
                fire_stores(gp, vb)

            fire_gather(g, vb)
        return carry

    lax.fori_loop(0, _ROUNDS, body, 0)

    for vb in range(_NBUF):
        gp = _PER_W - _NBUF + vb
        drain_stores(vb)
        drain_gather(vb)
        ep_lo = idx_v[pl.ds(2 * _CIDX, 16)]
        transpose_block(
            vb, ep_lo, ep_lo + 16, jax.lax.shift_right_logical(ep_lo, 31)
        )
        fire_stores(gp, vb)
    for vb in range(_NBUF):
        drain_stores(vb)


def kernel(token_ids, weight):
    idx_hm = jnp.transpose(token_ids).reshape(_TOT)
    out5 = _emb_gather(idx_hm, weight)
    return out5.transpose(2, 4, 0, 1, 3).reshape(_B, _H, _D)


# speedup vs baseline: 1.0117x; 1.0021x over previous
import functools

import jax
import jax.numpy as jnp
from jax import lax
from jax.experimental import pallas as pl
from jax.experimental.pallas import tpu as pltpu
from jax.experimental.pallas import tpu_sc as plsc

_NUM_EMB = 1000000
_D = 32
_B = 16384
_H = 200

_TOT = _B * _H
_NC, _NS = 2, 16
_NW = _NC * _NS
_BT = _B // 128
_NTILE = _H * _BT
_PER_W = _NTILE // _NW
_NBUF = 4
_ROUNDS = _PER_W // _NBUF
_CHUNK = 80
_CIDX = _CHUNK * 128


@functools.partial(
    pl.kernel,
    mesh=plsc.VectorSubcoreMesh(core_axis_name="c", subcore_axis_name="s"),
    out_type=jax.ShapeDtypeStruct((_H, _D // 8, _BT, 8, 128), jnp.float32),
    compiler_params=pltpu.CompilerParams(
        use_tc_tiling_on_sc=False, needs_layout_passes=False
    ),
    scratch_types=[
        pltpu.VMEM((2 * _CHUNK + 1, 128), jnp.int32),
        pltpu.VMEM((_NBUF, 128, _D), jnp.float32),
        pltpu.VMEM((_NBUF, _D, 137), jnp.float32),
    ]
    + [pltpu.SemaphoreType.DMA] * (2 * _NBUF),
)
def _emb_gather(idx_hbm, tab_hbm, out_hbm, idx_v, rows_v, trans_v, *sems):
    sem_g = sems[:_NBUF]
    sem_o = sems[_NBUF:]
    wid = lax.axis_index("s") * _NC + lax.axis_index("c")
    base = wid * _PER_W

    idx_v[2 * _CHUNK, pl.ds(0, 16)] = jnp.arange(16, dtype=jnp.int32)

    def load_chunk(g):
        c = g // _CHUNK
        src = pl.multiple_of(base + c * _CHUNK, 8)
        dst = pl.multiple_of((c % 2) * _CHUNK, 8)
        pltpu.sync_copy(
            idx_hbm.at[pl.ds(src, _CHUNK)], idx_v.at[pl.ds(dst, _CHUNK)]
        )

    def fire_gather(g, b):
        pltpu.async_copy(
            tab_hbm.at[idx_v.at[g % (2 * _CHUNK)]], rows_v.at[b], sem_g[b]
        )

    def drain_gather(b):
        pltpu.make_async_copy(
            tab_hbm.at[idx_v.at[0]], rows_v.at[b], sem_g[b]
        ).wait()

    def transpose_block(b, r_lo, r_hi, czero):
        tr_ref = trans_v.at[b]
        for j in range(128):
            lo = rows_v[b, j, pl.ds(0, 16)]
            hi = rows_v[b, j, pl.ds(16, 16)]
            cj = czero + j
            plsc.store_scatter(tr_ref, [r_lo, cj], lo)
            plsc.store_scatter(tr_ref, [r_hi, cj], hi)

    def fire_stores(gp, b):
        e = base + gp
        h = (e // 1024) * 8 + e % 8
        tc = (e // 8) % 128
        for tr in range(_D // 8):
            pltpu.async_copy(
                trans_v.at[b, pl.ds(tr * 8, 8), pl.ds(0, 128)],
                out_hbm.at[h, tr, tc],
                sem_o[b],
            )

    def drain_stores(b):
        for _ in range(_D // 8):
            pltpu.make_async_copy(
                trans_v.at[b, pl.ds(0, 8), pl.ds(0, 128)],
                out_hbm.at[0, 0, 0],
                sem_o[b],
            ).wait()

    def body(r, carry):
        g0 = r * _NBUF
        r_lo = idx_v[2 * _CHUNK, pl.ds(0, 16)]
        czero = jax.lax.shift_right_logical(r_lo, 31)
        r_hi = r_lo + 16

        @pl.when(g0 % _CHUNK == 0)
        def _():
            load_chunk(g0)

        for vb in range(_NBUF):
            g = g0 + vb
            gp = g - _NBUF

            @pl.when(g >= 2 * _NBUF)
            def _():
                drain_stores(vb)

            @pl.when(g >= _NBUF)
            def _():
                drain_gather(vb)
                transpose_block(vb, r_lo, r_hi, czero)
                fire_stores(gp, vb)

            fire_gather(g, vb)
        return carry

    lax.fori_loop(0, _ROUNDS, body, 0)

    for vb in range(_NBUF):
        gp = _PER_W - _NBUF + vb
        drain_stores(vb)
        drain_gather(vb)
        ep_lo = idx_v[2 * _CHUNK, pl.ds(0, 16)]
        transpose_block(
            vb, ep_lo, ep_lo + 16, jax.lax.shift_right_logical(ep_lo, 31)
        )
        fire_stores(gp, vb)
    for vb in range(_NBUF):
        drain_stores(vb)


def kernel(token_ids, weight):
    idx4 = jnp.transpose(token_ids).reshape(_H // 8, 8, _BT, 128)
    idx2 = idx4.transpose(0, 2, 1, 3).reshape(_NTILE, 128)
    out5 = _emb_gather(idx2, weight)
    return out5.transpose(2, 4, 0, 1, 3).reshape(_B, _H, _D)
